# baseline (speedup 1.0000x reference)
import jax
import jax.numpy as jnp
from jax.experimental import pallas as pl
from jax.experimental.pallas import tpu as pltpu

N = 10000
E = 320000
B = 64
RATIO = 0.5


def _gcn(x, row, col, ew, W, b):
    h = x @ W
    sl = jnp.arange(N, dtype=row.dtype)
    r2 = jnp.concatenate([row, sl])
    c2 = jnp.concatenate([col, sl])
    w2 = jnp.concatenate([ew, jnp.ones((N,), ew.dtype)])
    deg = jax.ops.segment_sum(w2, r2, num_segments=N)
    dis = 1.0 / jnp.sqrt(jnp.maximum(deg, 1e-12))
    norm = dis[r2] * w2 * dis[c2]
    out = jax.ops.segment_sum(norm[:, None] * h[c2], r2, num_segments=N)
    return out + b


def _sagpool(x, row, col, ew, batch, mask, Ws, bs):
    score = _gcn(x, row, col, ew, Ws, bs)[:, 0]
    score = jnp.where(mask, score, -1e30)
    order = jnp.lexsort((-score, batch))
    counts = jax.ops.segment_sum(jnp.ones((N,), jnp.int32), batch, num_segments=B)
    starts = jnp.concatenate([jnp.zeros((1,), jnp.int32), jnp.cumsum(counts)[:-1]])
    pos_in = jnp.arange(N, dtype=jnp.int32) - starts[batch[order]]
    rank = jnp.zeros((N,), jnp.int32).at[order].set(pos_in)
    alive = jax.ops.segment_sum(mask.astype(jnp.float32), batch, num_segments=B)
    k = jnp.ceil(RATIO * alive)
    new_mask = (rank.astype(jnp.float32) < k[batch]) & mask
    attn = jnp.tanh(score)
    xn = x * attn[:, None] * new_mask[:, None].astype(x.dtype)
    ewn = ew * new_mask[row].astype(ew.dtype) * new_mask[col].astype(ew.dtype)
    return xn, ewn, new_mask


def _readout(x, batch, mask):
    xm = jnp.where(mask[:, None], x, -1e30)
    mx = jax.ops.segment_max(xm, batch, num_segments=B)
    cnt = jax.ops.segment_sum(mask.astype(x.dtype), batch, num_segments=B)
    s = jax.ops.segment_sum(x * mask[:, None].astype(x.dtype), batch, num_segments=B)
    mean = s / jnp.maximum(cnt, 1.0)[:, None]
    mx = jnp.where(cnt[:, None] > 0, mx, 0.0)
    return jnp.concatenate([mx, mean], axis=1)


def _conv2d(x, w, b, pad):
    out = jax.lax.conv_general_dilated(
        x, w, (1, 1), [(pad, pad), (pad, pad)],
        dimension_numbers=("NCHW", "OIHW", "NCHW"))
    return out + b[None, :, None, None]


def _avgpool2(x):
    b, c, h, w = x.shape
    return x.reshape(b, c, h // 2, 2, w // 2, 2).mean(axis=(3, 5))


def _mlp_kernel(gc_ref, o_ref, lw1a_ref, lw1b_ref, lb1_ref, lw2_ref, lb2_ref, out_ref):
    z = gc_ref[...] @ lw1a_ref[...] + o_ref[...] @ lw1b_ref[...] + lb1_ref[...]
    z = jax.nn.relu(z)
    out_ref[...] = jax.nn.sigmoid(z @ lw2_ref[...] + lb2_ref[...])


def kernel(x, pos, edge_attr, strata_data, W1, b1, Ws1, bs1, W2, b2, Ws2, bs2,
           W3, b3, Ws3, bs3, cw1, cb1, cw2, cb2, cw3, cb3, gamma, beta,
           lw1, lb1, lw2, lb2, edge_index, batch):
    row = edge_index[0]
    col = edge_index[1]
    xf = jnp.concatenate([x, pos], axis=1)
    h = jax.nn.relu(_gcn(xf, row, col, edge_attr, W1, b1))
    mask = jnp.ones((N,), dtype=bool)
    h, ew, mask = _sagpool(h, row, col, edge_attr, batch, mask, Ws1, bs1)
    x1 = _readout(h, batch, mask)
    h = jax.nn.relu(_gcn(h, row, col, ew, W2, b2))
    h, ew, mask = _sagpool(h, row, col, ew, batch, mask, Ws2, bs2)
    x2 = _readout(h, batch, mask)
    h = jax.nn.relu(_gcn(h, row, col, ew, W3, b3))
    h, ew, mask = _sagpool(h, row, col, ew, batch, mask, Ws3, bs3)
    x3 = _readout(h, batch, mask)
    gc = x1 + x2 + x3

    o = jax.nn.relu(_conv2d(strata_data, cw1, cb1, 2)); o = _avgpool2(o)
    o = jax.nn.relu(_conv2d(o, cw2, cb2, 1)); o = _avgpool2(o)
    o = jax.nn.relu(_conv2d(o, cw3, cb3, 1)); o = _avgpool2(o)
    mu = o.mean(axis=(0, 2, 3), keepdims=True)
    var = o.var(axis=(0, 2, 3), keepdims=True)
    o = (o - mu) / jnp.sqrt(var + 1e-5) * gamma[None, :, None, None] + beta[None, :, None, None]
    o = o.reshape(o.shape[0], -1)

    out = pl.pallas_call(
        _mlp_kernel,
        out_shape=jax.ShapeDtypeStruct((B, 1), jnp.float32),
    )(gc, o, lw1[:256], lw1[256:], lb1[None, :], lw2, lb2[None, :])
    return out


# speedup vs baseline: 12.5183x; 12.5183x over previous
import jax
import jax.numpy as jnp
from jax import lax
from jax.experimental import pallas as pl
from jax.experimental.pallas import tpu as pltpu
from jax.experimental.pallas import tpu_sc as plsc

N = 10000
E = 320000
B = 64
RATIO = 0.5
F = 128

NC = 2
NS = 16
NW = NC * NS
EPW = E // NW
CH = 80
SGE = 2000
CPS = SGE // CH
NSTAGE = EPW // SGE
NP = 10240
NSL = NP // NS


def _wid(c, s):
    return s * NC + c


def _deg_body(row_hbm, ew_hbm, zn_hbm, out_hbm, ewst, rowbuf, acc):
    c = lax.axis_index("c")
    s = lax.axis_index("s")
    pltpu.sync_copy(zn_hbm.at[pl.ds(s * NSL, NSL)], acc.at[pl.ds(s * NSL, NSL)])
    plsc.subcore_barrier()
    base = _wid(c, s) * EPW

    def outer(t, carry):
        e0 = pl.multiple_of(base + t * SGE, 8)
        pltpu.sync_copy(ew_hbm.at[pl.ds(e0, SGE)], ewst)

        def chunk(k, carry2):
            o = pl.multiple_of(k * CH, 8)
            pltpu.sync_copy(row_hbm.at[pl.ds(e0 + o, CH)], rowbuf)
            pltpu.sync_copy(ewst.at[pl.ds(o, CH)], acc.at[rowbuf], add=True)
            return carry2

        return lax.fori_loop(0, CPS, chunk, carry)

    lax.fori_loop(0, NSTAGE, outer, 0)
    plsc.subcore_barrier()
    pltpu.sync_copy(acc.at[pl.ds(s * NSL, NSL)], out_hbm.at[c, pl.ds(s * NSL, NSL)])


_deg_call = pl.kernel(
    _deg_body,
    out_type=jax.ShapeDtypeStruct((NC, NP), jnp.float32),
    mesh=plsc.VectorSubcoreMesh(core_axis_name="c", subcore_axis_name="s"),
    compiler_params=pltpu.CompilerParams(needs_layout_passes=False),
    scratch_types=[
        pltpu.VMEM((SGE,), jnp.float32),
        pltpu.VMEM((CH,), jnp.int32),
        pltpu.VMEM_SHARED((NP,), jnp.float32),
    ],
)


def _agg_body(h_hbm, dis_hbm, col_hbm, row_hbm, ew_hbm, zf_hbm, out_hbm,
              colst, ewst, rowbuf, disv, wbuf, rows_v, acc, sem):
    c = lax.axis_index("c")
    s = lax.axis_index("s")
    pltpu.sync_copy(zf_hbm.at[pl.ds(s * NSL, NSL)], acc.at[pl.ds(s * NSL, NSL)])
    pltpu.sync_copy(dis_hbm, disv)
    plsc.subcore_barrier()
    base = _wid(c, s) * EPW

    def outer(t, carry):
        e0 = pl.multiple_of(base + t * SGE, 8)
        pltpu.sync_copy(col_hbm.at[pl.ds(e0, SGE)], colst)
        pltpu.sync_copy(ew_hbm.at[pl.ds(e0, SGE)], ewst)

        def chunk(k, carry2):
            o = pl.multiple_of(k * CH, 8)
            pltpu.sync_copy(row_hbm.at[pl.ds(e0 + o, CH)], rowbuf)
            pltpu.async_copy(
                h_hbm.at[colst.at[pl.ds(o, CH)]], rows_v, sem).wait()
            for g in range(CH // 16):
                c16 = colst[pl.ds(o + g * 16, 16)]
                dv = plsc.load_gather(disv, [c16])
                wbuf[pl.ds(g * 16, 16)] = dv * ewst[pl.ds(o + g * 16, 16)]

            def scale(e, carry3):
                wspl = plsc.load_gather(
                    wbuf, [jnp.zeros((16,), jnp.int32) + e])
                for f in range(F // 16):
                    rows_v[e, pl.ds(f * 16, 16)] = (
                        rows_v[e, pl.ds(f * 16, 16)] * wspl)
                return carry3

            lax.fori_loop(0, CH, scale, 0)
            pltpu.sync_copy(rows_v, acc.at[rowbuf], add=True)
            return carry2

        return lax.fori_loop(0, CPS, chunk, carry)

    lax.fori_loop(0, NSTAGE, outer, 0)
    plsc.subcore_barrier()
    pltpu.sync_copy(acc.at[pl.ds(s * NSL, NSL)], out_hbm.at[c, pl.ds(s * NSL, NSL)])


_agg_call = pl.kernel(
    _agg_body,
    out_type=jax.ShapeDtypeStruct((NC, NP, F), jnp.float32),
    mesh=plsc.VectorSubcoreMesh(core_axis_name="c", subcore_axis_name="s"),
    compiler_params=pltpu.CompilerParams(needs_layout_passes=False),
    scratch_types=[
        pltpu.VMEM((SGE,), jnp.int32),
        pltpu.VMEM((SGE,), jnp.float32),
        pltpu.VMEM((CH,), jnp.int32),
        pltpu.VMEM((NP,), jnp.float32),
        pltpu.VMEM((CH,), jnp.float32),
        pltpu.VMEM((CH, F), jnp.float32),
        pltpu.VMEM_SHARED((NP, F), jnp.float32),
        pltpu.SemaphoreType.DMA,
    ],
)


def _score_body(vg_hbm, col_hbm, row_hbm, ew_hbm, zn_hbm, out_hbm,
                colst, ewst, rowbuf, vgv, vout, acc):
    c = lax.axis_index("c")
    s = lax.axis_index("s")
    pltpu.sync_copy(zn_hbm.at[pl.ds(s * NSL, NSL)], acc.at[pl.ds(s * NSL, NSL)])
    pltpu.sync_copy(vg_hbm, vgv)
    plsc.subcore_barrier()
    base = _wid(c, s) * EPW

    def outer(t, carry):
        e0 = pl.multiple_of(base + t * SGE, 8)
        pltpu.sync_copy(col_hbm.at[pl.ds(e0, SGE)], colst)
        pltpu.sync_copy(ew_hbm.at[pl.ds(e0, SGE)], ewst)

        def chunk(k, carry2):
            o = pl.multiple_of(k * CH, 8)
            pltpu.sync_copy(row_hbm.at[pl.ds(e0 + o, CH)], rowbuf)
            for g in range(CH // 16):
                c16 = colst[pl.ds(o + g * 16, 16)]
                vv = plsc.load_gather(vgv, [c16])
                vout[pl.ds(g * 16, 16)] = vv * ewst[pl.ds(o + g * 16, 16)]
            pltpu.sync_copy(vout, acc.at[rowbuf], add=True)
            return carry2

        return lax.fori_loop(0, CPS, chunk, carry)

    lax.fori_loop(0, NSTAGE, outer, 0)
    plsc.subcore_barrier()
    pltpu.sync_copy(acc.at[pl.ds(s * NSL, NSL)], out_hbm.at[c, pl.ds(s * NSL, NSL)])


_score_call = pl.kernel(
    _score_body,
    out_type=jax.ShapeDtypeStruct((NC, NP), jnp.float32),
    mesh=plsc.VectorSubcoreMesh(core_axis_name="c", subcore_axis_name="s"),
    compiler_params=pltpu.CompilerParams(needs_layout_passes=False),
    scratch_types=[
        pltpu.VMEM((SGE,), jnp.int32),
        pltpu.VMEM((SGE,), jnp.float32),
        pltpu.VMEM((CH,), jnp.int32),
        pltpu.VMEM((NP,), jnp.float32),
        pltpu.VMEM((CH,), jnp.float32),
        pltpu.VMEM_SHARED((NP,), jnp.float32),
    ],
)


def _ewn_body(mf_hbm, col_hbm, row_hbm, ew_hbm, zn_hbm, ewn_hbm, degp_hbm,
              colst, rowst, ewst, rowbuf, mv, vout, acc):
    c = lax.axis_index("c")
    s = lax.axis_index("s")
    pltpu.sync_copy(zn_hbm.at[pl.ds(s * NSL, NSL)], acc.at[pl.ds(s * NSL, NSL)])
    pltpu.sync_copy(mf_hbm, mv)
    plsc.subcore_barrier()
    base = _wid(c, s) * EPW

    def outer(t, carry):
        e0 = pl.multiple_of(base + t * SGE, 8)
        pltpu.sync_copy(col_hbm.at[pl.ds(e0, SGE)], colst)
        pltpu.sync_copy(row_hbm.at[pl.ds(e0, SGE)], rowst)
        pltpu.sync_copy(ew_hbm.at[pl.ds(e0, SGE)], ewst)

        def chunk(k, carry2):
            o = pl.multiple_of(k * CH, 8)
            pltpu.sync_copy(row_hbm.at[pl.ds(e0 + o, CH)], rowbuf)
            for g in range(CH // 16):
                r16 = rowst[pl.ds(o + g * 16, 16)]
                c16 = colst[pl.ds(o + g * 16, 16)]
                mr = plsc.load_gather(mv, [r16])
                mc = plsc.load_gather(mv, [c16])
                vout[pl.ds(g * 16, 16)] = ewst[pl.ds(o + g * 16, 16)] * mr * mc
            pltpu.sync_copy(vout, ewn_hbm.at[pl.ds(e0 + o, CH)])
            pltpu.sync_copy(vout, acc.at[rowbuf], add=True)
            return carry2

        return lax.fori_loop(0, CPS, chunk, carry)

    lax.fori_loop(0, NSTAGE, outer, 0)
    plsc.subcore_barrier()
    pltpu.sync_copy(acc.at[pl.ds(s * NSL, NSL)], degp_hbm.at[c, pl.ds(s * NSL, NSL)])


_ewn_call = pl.kernel(
    _ewn_body,
    out_type=[
        jax.ShapeDtypeStruct((E,), jnp.float32),
        jax.ShapeDtypeStruct((NC, NP), jnp.float32),
    ],
    mesh=plsc.VectorSubcoreMesh(core_axis_name="c", subcore_axis_name="s"),
    compiler_params=pltpu.CompilerParams(needs_layout_passes=False),
    scratch_types=[
        pltpu.VMEM((SGE,), jnp.int32),
        pltpu.VMEM((SGE,), jnp.int32),
        pltpu.VMEM((SGE,), jnp.float32),
        pltpu.VMEM((CH,), jnp.int32),
        pltpu.VMEM((NP,), jnp.float32),
        pltpu.VMEM((CH,), jnp.float32),
        pltpu.VMEM_SHARED((NP,), jnp.float32),
    ],
)


def _mlp_kernel(gc_ref, o_ref, lw1a_ref, lw1b_ref, lb1_ref, lw2_ref, lb2_ref, out_ref):
    z = gc_ref[...] @ lw1a_ref[...] + o_ref[...] @ lw1b_ref[...] + lb1_ref[...]
    z = jax.nn.relu(z)
    out_ref[...] = jax.nn.sigmoid(z @ lw2_ref[...] + lb2_ref[...])


def _rank_pool(score, maskf, batch):
    mask = maskf > 0.5
    smask = jnp.where(mask, score, -1e30)
    idx = jnp.arange(N, dtype=jnp.int32)
    beq = batch[:, None] == batch[None, :]
    better = (smask[None, :] > smask[:, None]) | (
        (smask[None, :] == smask[:, None]) & (idx[None, :] < idx[:, None]))
    rank = jnp.sum(beq & better, axis=1).astype(jnp.float32)
    alive = jax.ops.segment_sum(maskf, batch, num_segments=B)
    k = jnp.ceil(RATIO * alive)
    new_mask = (rank < k[batch]) & mask
    return smask, new_mask


def _readout(x, batch, mask):
    xm = jnp.where(mask[:, None], x, -1e30)
    mx = jax.ops.segment_max(xm, batch, num_segments=B)
    cnt = jax.ops.segment_sum(mask.astype(x.dtype), batch, num_segments=B)
    s = jax.ops.segment_sum(x * mask[:, None].astype(x.dtype), batch, num_segments=B)
    mean = s / jnp.maximum(cnt, 1.0)[:, None]
    mx = jnp.where(cnt[:, None] > 0, mx, 0.0)
    return jnp.concatenate([mx, mean], axis=1)


def _conv2d(x, w, b, pad):
    out = jax.lax.conv_general_dilated(
        x, w, (1, 1), [(pad, pad), (pad, pad)],
        dimension_numbers=("NCHW", "OIHW", "NCHW"))
    return out + b[None, :, None, None]


def _avgpool2(x):
    b, c, h, w = x.shape
    return x.reshape(b, c, h // 2, 2, w // 2, 2).mean(axis=(3, 5))


def kernel(x, pos, edge_attr, strata_data, W1, b1, Ws1, bs1, W2, b2, Ws2, bs2,
           W3, b3, Ws3, bs3, cw1, cb1, cw2, cb2, cw3, cb3, gamma, beta,
           lw1, lb1, lw2, lb2, edge_index, batch):
    row = edge_index[0]
    col = edge_index[1]
    ew = edge_attr
    zn = jnp.zeros((NP,), jnp.float32)
    zf = jnp.zeros((NP, F), jnp.float32)

    h = jnp.zeros((NP, F), jnp.float32).at[:N].set(
        jnp.concatenate([x, pos], axis=1))

    degP = _deg_call(row, ew, zn)
    dis = lax.rsqrt(jnp.maximum(degP[0] + degP[1] + 1.0, 1e-12))

    gc = jnp.zeros((B, 2 * F), jnp.float32)
    weights = [(W1, b1, Ws1, bs1), (W2, b2, Ws2, bs2), (W3, b3, Ws3, bs3)]
    maskf = jnp.ones((N,), jnp.float32)
    for r in range(3):
        W, b, Ws, bs = weights[r]
        P = _agg_call(h, dis, col, row, ew, zf)
        agg = dis[:, None] * (P[0] + P[1]) + (dis * dis)[:, None] * h
        hp = jax.nn.relu(agg @ W + b)
        v = (hp @ Ws)[:, 0]
        vg = dis * v
        sP = _score_call(vg, col, row, ew, zn)
        score = (dis * (sP[0] + sP[1] + vg))[:N] + bs[0]
        smask, nm = _rank_pool(score, maskf, batch)
        nmf = nm.astype(jnp.float32)
        xn = hp[:N] * jnp.tanh(smask)[:, None] * nmf[:, None]
        gc = gc + _readout(xn, batch, nm)
        h = jnp.zeros((NP, F), jnp.float32).at[:N].set(xn)
        maskf = nmf
        if r < 2:
            nmp = jnp.zeros((NP,), jnp.float32).at[:N].set(nmf)
            ew, degP = _ewn_call(nmp, col, row, ew, zn)
            dis = lax.rsqrt(jnp.maximum(degP[0] + degP[1] + 1.0, 1e-12))

    o = jax.nn.relu(_conv2d(strata_data, cw1, cb1, 2)); o = _avgpool2(o)
    o = jax.nn.relu(_conv2d(o, cw2, cb2, 1)); o = _avgpool2(o)
    o = jax.nn.relu(_conv2d(o, cw3, cb3, 1)); o = _avgpool2(o)
    mu = o.mean(axis=(0, 2, 3), keepdims=True)
    var = o.var(axis=(0, 2, 3), keepdims=True)
    o = (o - mu) / jnp.sqrt(var + 1e-5) * gamma[None, :, None, None] + beta[None, :, None, None]
    o = o.reshape(o.shape[0], -1)

    out = pl.pallas_call(
        _mlp_kernel,
        out_shape=jax.ShapeDtypeStruct((B, 1), jnp.float32),
    )(gc, o, lw1[:256], lw1[256:], lb1[None, :], lw2, lb2[None, :])
    return out


# speedup vs baseline: 13.6078x; 1.0870x over previous
import jax
import jax.numpy as jnp
from jax import lax
from jax.experimental import pallas as pl
from jax.experimental.pallas import tpu as pltpu
from jax.experimental.pallas import tpu_sc as plsc

N = 10000
E = 320000
B = 64
RATIO = 0.5
F = 128

NC = 2
NS = 16
NW = NC * NS
EPW = E // NW
CH = 80
SGE = 2000
CPS = SGE // CH
NSTAGE = EPW // SGE
NP = 10240
NSL = NP // NS


def _wid(c, s):
    return s * NC + c


def _deg_body(row_hbm, ew_hbm, zn_hbm, out_hbm, ewst, rowbuf, acc):
    c = lax.axis_index("c")
    s = lax.axis_index("s")
    pltpu.sync_copy(zn_hbm.at[pl.ds(s * NSL, NSL)], acc.at[pl.ds(s * NSL, NSL)])
    plsc.subcore_barrier()
    base = _wid(c, s) * EPW

    def outer(t, carry):
        e0 = pl.multiple_of(base + t * SGE, 8)
        pltpu.sync_copy(ew_hbm.at[pl.ds(e0, SGE)], ewst)

        def chunk(k, carry2):
            o = pl.multiple_of(k * CH, 8)
            pltpu.sync_copy(row_hbm.at[pl.ds(e0 + o, CH)], rowbuf)
            pltpu.sync_copy(ewst.at[pl.ds(o, CH)], acc.at[rowbuf], add=True)
            return carry2

        return lax.fori_loop(0, CPS, chunk, carry)

    lax.fori_loop(0, NSTAGE, outer, 0)
    plsc.subcore_barrier()
    pltpu.sync_copy(acc.at[pl.ds(s * NSL, NSL)], out_hbm.at[c, pl.ds(s * NSL, NSL)])


_deg_call = pl.kernel(
    _deg_body,
    out_type=jax.ShapeDtypeStruct((NC, NP), jnp.float32),
    mesh=plsc.VectorSubcoreMesh(core_axis_name="c", subcore_axis_name="s"),
    compiler_params=pltpu.CompilerParams(needs_layout_passes=False),
    scratch_types=[
        pltpu.VMEM((SGE,), jnp.float32),
        pltpu.VMEM((CH,), jnp.int32),
        pltpu.VMEM_SHARED((NP,), jnp.float32),
    ],
)


def _agg_body(h_hbm, dis_hbm, col_hbm, row_hbm, ew_hbm, zf_hbm, out_hbm,
              colst, ewst, rowbuf, disv, wbuf, rows_v, acc, sem):
    c = lax.axis_index("c")
    s = lax.axis_index("s")
    pltpu.sync_copy(zf_hbm.at[pl.ds(s * NSL, NSL)], acc.at[pl.ds(s * NSL, NSL)])
    pltpu.sync_copy(dis_hbm, disv)
    plsc.subcore_barrier()
    base = _wid(c, s) * EPW

    def outer(t, carry):
        e0 = pl.multiple_of(base + t * SGE, 8)
        pltpu.sync_copy(col_hbm.at[pl.ds(e0, SGE)], colst)
        pltpu.sync_copy(ew_hbm.at[pl.ds(e0, SGE)], ewst)

        def chunk(k, carry2):
            o = pl.multiple_of(k * CH, 8)
            pltpu.sync_copy(row_hbm.at[pl.ds(e0 + o, CH)], rowbuf)
            pltpu.async_copy(
                h_hbm.at[colst.at[pl.ds(o, CH)]], rows_v, sem).wait()
            for g in range(CH // 16):
                c16 = colst[pl.ds(o + g * 16, 16)]
                dv = plsc.load_gather(disv, [c16])
                wbuf[pl.ds(g * 16, 16)] = dv * ewst[pl.ds(o + g * 16, 16)]

            def scale(e, carry3):
                wspl = plsc.load_gather(
                    wbuf, [jnp.zeros((16,), jnp.int32) + e])
                for f in range(F // 16):
                    rows_v[e, pl.ds(f * 16, 16)] = (
                        rows_v[e, pl.ds(f * 16, 16)] * wspl)
                return carry3

            lax.fori_loop(0, CH, scale, 0)
            pltpu.sync_copy(rows_v, acc.at[rowbuf], add=True)
            return carry2

        return lax.fori_loop(0, CPS, chunk, carry)

    lax.fori_loop(0, NSTAGE, outer, 0)
    plsc.subcore_barrier()
    pltpu.sync_copy(acc.at[pl.ds(s * NSL, NSL)], out_hbm.at[c, pl.ds(s * NSL, NSL)])


_agg_call = pl.kernel(
    _agg_body,
    out_type=jax.ShapeDtypeStruct((NC, NP, F), jnp.float32),
    mesh=plsc.VectorSubcoreMesh(core_axis_name="c", subcore_axis_name="s"),
    compiler_params=pltpu.CompilerParams(needs_layout_passes=False),
    scratch_types=[
        pltpu.VMEM((SGE,), jnp.int32),
        pltpu.VMEM((SGE,), jnp.float32),
        pltpu.VMEM((CH,), jnp.int32),
        pltpu.VMEM((NP,), jnp.float32),
        pltpu.VMEM((CH,), jnp.float32),
        pltpu.VMEM((CH, F), jnp.float32),
        pltpu.VMEM_SHARED((NP, F), jnp.float32),
        pltpu.SemaphoreType.DMA,
    ],
)


def _score_body(vg_hbm, col_hbm, row_hbm, ew_hbm, zn_hbm, out_hbm,
                colst, ewst, rowbuf, vgv, vout, acc):
    c = lax.axis_index("c")
    s = lax.axis_index("s")
    pltpu.sync_copy(zn_hbm.at[pl.ds(s * NSL, NSL)], acc.at[pl.ds(s * NSL, NSL)])
    pltpu.sync_copy(vg_hbm, vgv)
    plsc.subcore_barrier()
    base = _wid(c, s) * EPW

    def outer(t, carry):
        e0 = pl.multiple_of(base + t * SGE, 8)
        pltpu.sync_copy(col_hbm.at[pl.ds(e0, SGE)], colst)
        pltpu.sync_copy(ew_hbm.at[pl.ds(e0, SGE)], ewst)

        def chunk(k, carry2):
            o = pl.multiple_of(k * CH, 8)
            pltpu.sync_copy(row_hbm.at[pl.ds(e0 + o, CH)], rowbuf)
            for g in range(CH // 16):
                c16 = colst[pl.ds(o + g * 16, 16)]
                vv = plsc.load_gather(vgv, [c16])
                vout[pl.ds(g * 16, 16)] = vv * ewst[pl.ds(o + g * 16, 16)]
            pltpu.sync_copy(vout, acc.at[rowbuf], add=True)
            return carry2

        return lax.fori_loop(0, CPS, chunk, carry)

    lax.fori_loop(0, NSTAGE, outer, 0)
    plsc.subcore_barrier()
    pltpu.sync_copy(acc.at[pl.ds(s * NSL, NSL)], out_hbm.at[c, pl.ds(s * NSL, NSL)])


_score_call = pl.kernel(
    _score_body,
    out_type=jax.ShapeDtypeStruct((NC, NP), jnp.float32),
    mesh=plsc.VectorSubcoreMesh(core_axis_name="c", subcore_axis_name="s"),
    compiler_params=pltpu.CompilerParams(needs_layout_passes=False),
    scratch_types=[
        pltpu.VMEM((SGE,), jnp.int32),
        pltpu.VMEM((SGE,), jnp.float32),
        pltpu.VMEM((CH,), jnp.int32),
        pltpu.VMEM((NP,), jnp.float32),
        pltpu.VMEM((CH,), jnp.float32),
        pltpu.VMEM_SHARED((NP,), jnp.float32),
    ],
)


def _ewn_body(mf_hbm, col_hbm, row_hbm, ew_hbm, zn_hbm, ewn_hbm, degp_hbm,
              colst, rowst, ewst, rowbuf, mv, vout, acc):
    c = lax.axis_index("c")
    s = lax.axis_index("s")
    pltpu.sync_copy(zn_hbm.at[pl.ds(s * NSL, NSL)], acc.at[pl.ds(s * NSL, NSL)])
    pltpu.sync_copy(mf_hbm, mv)
    plsc.subcore_barrier()
    base = _wid(c, s) * EPW

    def outer(t, carry):
        e0 = pl.multiple_of(base + t * SGE, 8)
        pltpu.sync_copy(col_hbm.at[pl.ds(e0, SGE)], colst)
        pltpu.sync_copy(row_hbm.at[pl.ds(e0, SGE)], rowst)
        pltpu.sync_copy(ew_hbm.at[pl.ds(e0, SGE)], ewst)

        def chunk(k, carry2):
            o = pl.multiple_of(k * CH, 8)
            pltpu.sync_copy(row_hbm.at[pl.ds(e0 + o, CH)], rowbuf)
            for g in range(CH // 16):
                r16 = rowst[pl.ds(o + g * 16, 16)]
                c16 = colst[pl.ds(o + g * 16, 16)]
                mr = plsc.load_gather(mv, [r16])
                mc = plsc.load_gather(mv, [c16])
                vout[pl.ds(g * 16, 16)] = ewst[pl.ds(o + g * 16, 16)] * mr * mc
            pltpu.sync_copy(vout, ewn_hbm.at[pl.ds(e0 + o, CH)])
            pltpu.sync_copy(vout, acc.at[rowbuf], add=True)
            return carry2

        return lax.fori_loop(0, CPS, chunk, carry)

    lax.fori_loop(0, NSTAGE, outer, 0)
    plsc.subcore_barrier()
    pltpu.sync_copy(acc.at[pl.ds(s * NSL, NSL)], degp_hbm.at[c, pl.ds(s * NSL, NSL)])


_ewn_call = pl.kernel(
    _ewn_body,
    out_type=[
        jax.ShapeDtypeStruct((E,), jnp.float32),
        jax.ShapeDtypeStruct((NC, NP), jnp.float32),
    ],
    mesh=plsc.VectorSubcoreMesh(core_axis_name="c", subcore_axis_name="s"),
    compiler_params=pltpu.CompilerParams(needs_layout_passes=False),
    scratch_types=[
        pltpu.VMEM((SGE,), jnp.int32),
        pltpu.VMEM((SGE,), jnp.int32),
        pltpu.VMEM((SGE,), jnp.float32),
        pltpu.VMEM((CH,), jnp.int32),
        pltpu.VMEM((NP,), jnp.float32),
        pltpu.VMEM((CH,), jnp.float32),
        pltpu.VMEM_SHARED((NP,), jnp.float32),
    ],
)


TI = 1024
NBI = NP // TI
TJ = 2048
NBJ = NP // TJ


def _dis_body(degp_ref, dis_ref):
    d = degp_ref[...]
    dis_ref[...] = lax.rsqrt(jnp.maximum(d[0:1, :] + d[1:2, :] + 1.0, 1e-12))


def _dis_call(degP):
    return pl.pallas_call(
        _dis_body,
        out_shape=jax.ShapeDtypeStruct((1, NP), jnp.float32),
    )(degP)


def _gcnmm_body(p_ref, h_ref, dis_ref, w_ref, b_ref, ws_ref, hp_ref, vg_ref):
    dis = dis_ref[...]
    p = p_ref[...]
    g = dis * (p[0] + p[1]) + (dis * dis) * h_ref[...]
    hp = jax.nn.relu(
        jnp.dot(g, w_ref[...], preferred_element_type=jnp.float32) + b_ref[...])
    hp_ref[...] = hp
    vg_ref[...] = dis * jnp.dot(hp, ws_ref[...],
                                preferred_element_type=jnp.float32)


def _gcnmm_call(P, h, dis_col, W, b, Ws):
    return pl.pallas_call(
        _gcnmm_body,
        grid=(NBI,),
        in_specs=[
            pl.BlockSpec((NC, TI, F), lambda i: (0, i, 0)),
            pl.BlockSpec((TI, F), lambda i: (i, 0)),
            pl.BlockSpec((TI, 1), lambda i: (i, 0)),
            pl.BlockSpec((F, F), lambda i: (0, 0)),
            pl.BlockSpec((1, F), lambda i: (0, 0)),
            pl.BlockSpec((F, 1), lambda i: (0, 0)),
        ],
        out_specs=[
            pl.BlockSpec((TI, F), lambda i: (i, 0)),
            pl.BlockSpec((TI, 1), lambda i: (i, 0)),
        ],
        out_shape=[
            jax.ShapeDtypeStruct((NP, F), jnp.float32),
            jax.ShapeDtypeStruct((NP, 1), jnp.float32),
        ],
    )(P, h, dis_col, W, b, Ws)


def _smask_body(sp_ref, vg_ref, dis_ref, mf_ref, bs_ref, sm_ref):
    sp = sp_ref[...]
    score = dis_ref[...] * (sp[0:1, :] + sp[1:2, :] + vg_ref[...]) + bs_ref[0, 0]
    sm_ref[...] = jnp.where(mf_ref[...] > 0.5, score, -1e30)


def _smask_call(sP, vg_row, dis_row, mf_row, bs):
    return pl.pallas_call(
        _smask_body,
        out_shape=jax.ShapeDtypeStruct((1, NP), jnp.float32),
    )(sP, vg_row, dis_row, mf_row, bs)


def _rank_body(si_ref, sj_ref, bi_ref, bj_ref, mfj_ref, rank_ref, alive_ref):
    i = pl.program_id(0)
    j = pl.program_id(1)
    si = si_ref[...]
    sj = sj_ref[...]
    bi = bi_ref[...]
    bj = bj_ref[...]
    ig = i * TI + lax.broadcasted_iota(jnp.int32, (TI, 1), 0)
    jg = j * TJ + lax.broadcasted_iota(jnp.int32, (1, TJ), 1)
    cond = (sj > si) | ((sj == si) & (jg < ig))
    add = ((bj == bi) & cond).astype(jnp.float32)
    part = jnp.sum(add, axis=1, keepdims=True)

    @pl.when(j == 0)
    def _():
        rank_ref[...] = jnp.zeros_like(rank_ref)

    rank_ref[...] += part

    @pl.when(jnp.logical_and(i == 0, j == 0))
    def _():
        alive_ref[...] = jnp.zeros_like(alive_ref)

    @pl.when(i == 0)
    def _():
        oh = (bj == lax.broadcasted_iota(jnp.int32, (B, 1), 0)).astype(jnp.float32)
        alive_ref[...] += jnp.sum(oh * mfj_ref[...], axis=1, keepdims=True)


def _rank_call(sm_col, sm_row, b_col, b_row, mf_row):
    return pl.pallas_call(
        _rank_body,
        grid=(NBI, NBJ),
        in_specs=[
            pl.BlockSpec((TI, 1), lambda i, j: (i, 0)),
            pl.BlockSpec((1, TJ), lambda i, j: (0, j)),
            pl.BlockSpec((TI, 1), lambda i, j: (i, 0)),
            pl.BlockSpec((1, TJ), lambda i, j: (0, j)),
            pl.BlockSpec((1, TJ), lambda i, j: (0, j)),
        ],
        out_specs=[
            pl.BlockSpec((TI, 1), lambda i, j: (i, 0)),
            pl.BlockSpec((B, 1), lambda i, j: (0, 0)),
        ],
        out_shape=[
            jax.ShapeDtypeStruct((NP, 1), jnp.float32),
            jax.ShapeDtypeStruct((B, 1), jnp.float32),
        ],
    )(sm_col, sm_row, b_col, b_row, mf_row)


def _pool_body(rank_ref, alive_ref, sm_ref, mf_ref, b_ref, hp_ref,
               xn_ref, nmf_ref, sacc_ref, cnt_ref, mx_ref):
    i = pl.program_id(0)
    kb = jnp.ceil(RATIO * alive_ref[...])
    bi = b_ref[...]
    oh = (bi == lax.broadcasted_iota(jnp.int32, (1, B), 1)).astype(jnp.float32)
    kk = jnp.dot(oh, kb, preferred_element_type=jnp.float32)
    nm = jnp.logical_and(rank_ref[...] < kk, mf_ref[...] > 0.5)
    nmf = nm.astype(jnp.float32)
    xn = hp_ref[...] * jnp.tanh(sm_ref[...]) * nmf
    xn_ref[...] = xn
    nmf_ref[...] = nmf

    @pl.when(i == 0)
    def _():
        sacc_ref[...] = jnp.zeros_like(sacc_ref)
        cnt_ref[...] = jnp.zeros_like(cnt_ref)
        mx_ref[...] = jnp.full_like(mx_ref, -1e30)

    contract = (((0,), (0,)), ((), ()))
    sacc_ref[...] += lax.dot_general(oh, xn, contract,
                                     preferred_element_type=jnp.float32)
    cnt_ref[...] += lax.dot_general(oh, nmf, contract,
                                    preferred_element_type=jnp.float32)
    rows = []
    for bb in range(B):
        xm = jnp.where(jnp.logical_and(nm, bi == bb), xn, -1e30)
        rows.append(jnp.max(xm, axis=0, keepdims=True))
    mx_ref[...] = jnp.maximum(mx_ref[...], jnp.concatenate(rows, axis=0))


def _pool_call(rank, alive, sm_col, mf_col, b_col, hp):
    return pl.pallas_call(
        _pool_body,
        grid=(NBI,),
        in_specs=[
            pl.BlockSpec((TI, 1), lambda i: (i, 0)),
            pl.BlockSpec((B, 1), lambda i: (0, 0)),
            pl.BlockSpec((TI, 1), lambda i: (i, 0)),
            pl.BlockSpec((TI, 1), lambda i: (i, 0)),
            pl.BlockSpec((TI, 1), lambda i: (i, 0)),
            pl.BlockSpec((TI, F), lambda i: (i, 0)),
        ],
        out_specs=[
            pl.BlockSpec((TI, F), lambda i: (i, 0)),
            pl.BlockSpec((TI, 1), lambda i: (i, 0)),
            pl.BlockSpec((B, F), lambda i: (0, 0)),
            pl.BlockSpec((B, 1), lambda i: (0, 0)),
            pl.BlockSpec((B, F), lambda i: (0, 0)),
        ],
        out_shape=[
            jax.ShapeDtypeStruct((NP, F), jnp.float32),
            jax.ShapeDtypeStruct((NP, 1), jnp.float32),
            jax.ShapeDtypeStruct((B, F), jnp.float32),
            jax.ShapeDtypeStruct((B, 1), jnp.float32),
            jax.ShapeDtypeStruct((B, F), jnp.float32),
        ],
    )(rank, alive, sm_col, mf_col, b_col, hp)


def _convmm_body(x_ref, w_ref, b_ref, o_ref):
    o_ref[...] = jax.nn.relu(
        jnp.dot(x_ref[...], w_ref[...], preferred_element_type=jnp.float32)
        + b_ref[...])


def _convmm_call(patches, wmat, bvec):
    m, k = patches.shape
    co = wmat.shape[1]
    return pl.pallas_call(
        _convmm_body,
        out_shape=jax.ShapeDtypeStruct((m, co), jnp.float32),
    )(patches, wmat, bvec[None, :])


def _final_body(mx1, s1, c1, mx2, s2, c2, mx3, s3, c3, o_ref, g_ref,
                gmp_ref, btp_ref, a1_ref, a2_ref, b1o_ref, lb1_ref,
                lw2_ref, lb2_ref, out_ref):
    o = o_ref[...]
    mu = jnp.mean(o, axis=0, keepdims=True)
    m2 = jnp.mean(o * o, axis=0, keepdims=True)
    muc = jnp.dot(mu, g_ref[...], preferred_element_type=jnp.float32)
    m2c = jnp.dot(m2, g_ref[...], preferred_element_type=jnp.float32)
    var = m2c - muc * muc
    on = (o - muc) / jnp.sqrt(var + 1e-5) * gmp_ref[...] + btp_ref[...]

    z = lb1_ref[...] + jnp.dot(on, b1o_ref[...],
                               preferred_element_type=jnp.float32)
    for mx, s, c in ((mx1, s1, c1), (mx2, s2, c2), (mx3, s3, c3)):
        cnt = c[...]
        mxf = jnp.where(cnt > 0, mx[...], 0.0)
        mean = s[...] / jnp.maximum(cnt, 1.0)
        z += jnp.dot(mxf, a1_ref[...], preferred_element_type=jnp.float32)
        z += jnp.dot(mean, a2_ref[...], preferred_element_type=jnp.float32)
    z = jax.nn.relu(z)
    out_ref[...] = jax.nn.sigmoid(
        jnp.dot(z, lw2_ref[...], preferred_element_type=jnp.float32)
        + lb2_ref[...])


def _final_call(ro, o, G, gmp, btp, a1, a2, b1o, lb1, lw2, lb2):
    (mx1, s1, c1), (mx2, s2, c2), (mx3, s3, c3) = ro
    return pl.pallas_call(
        _final_body,
        out_shape=jax.ShapeDtypeStruct((B, 1), jnp.float32),
    )(mx1, s1, c1, mx2, s2, c2, mx3, s3, c3, o, G, gmp, btp,
      a1, a2, b1o, lb1[None, :], lw2, lb2[None, :])


def _im2col(xp, kh, kw):
    b, hp_, wp_, c = xp.shape
    h = hp_ - kh + 1
    w = wp_ - kw + 1
    cols = []
    for dy in range(kh):
        for dx in range(kw):
            cols.append(xp[:, dy:dy + h, dx:dx + w, :])
    pt = jnp.stack(cols, axis=3)
    return pt.reshape(b * h * w, kh * kw * c)


def _avgpool2_nhwc(x):
    b, h, w, c = x.shape
    return x.reshape(b, h // 2, 2, w // 2, 2, c).mean(axis=(2, 4))


def kernel(x, pos, edge_attr, strata_data, W1, b1, Ws1, bs1, W2, b2, Ws2, bs2,
           W3, b3, Ws3, bs3, cw1, cb1, cw2, cb2, cw3, cb3, gamma, beta,
           lw1, lb1, lw2, lb2, edge_index, batch):
    row = edge_index[0]
    col = edge_index[1]
    ew = edge_attr
    zn = jnp.zeros((NP,), jnp.float32)
    zf = jnp.zeros((NP, F), jnp.float32)

    h = jnp.zeros((NP, F), jnp.float32).at[:N].set(
        jnp.concatenate([x, pos], axis=1))
    batch_p = jnp.full((NP,), -1, jnp.int32).at[:N].set(batch)
    b_col = batch_p.reshape(NP, 1)
    b_row = batch_p.reshape(1, NP)
    mf_col = jnp.zeros((NP, 1), jnp.float32).at[:N].set(1.0)

    degP = _deg_call(row, ew, zn)
    dis_row = _dis_call(degP)
    dis_col = dis_row.reshape(NP, 1)

    ro = []
    weights = [(W1, b1, Ws1, bs1), (W2, b2, Ws2, bs2), (W3, b3, Ws3, bs3)]
    for r in range(3):
        W, b, Ws, bs = weights[r]
        P = _agg_call(h, dis_row.reshape(NP), col, row, ew, zf)
        hp, vg_col = _gcnmm_call(P, h, dis_col, W, b[None, :], Ws)
        sP = _score_call(vg_col.reshape(NP), col, row, ew, zn)
        sm_row = _smask_call(sP, vg_col.reshape(1, NP), dis_row,
                             mf_col.reshape(1, NP), bs[None, :])
        sm_col = sm_row.reshape(NP, 1)
        rank, alive = _rank_call(sm_col, sm_row, b_col, b_row,
                                 mf_col.reshape(1, NP))
        xn, nmf_col, sacc, cnt, mx = _pool_call(rank, alive, sm_col,
                                                mf_col, b_col, hp)
        ro.append((mx, sacc, cnt))
        h = xn
        mf_col = nmf_col
        if r < 2:
            ew, degP = _ewn_call(nmf_col.reshape(NP), col, row, ew, zn)
            dis_row = _dis_call(degP)
            dis_col = dis_row.reshape(NP, 1)

    x0p = jnp.pad(strata_data.transpose(0, 2, 3, 1),
                  ((0, 0), (2, 2), (2, 2), (0, 0)))
    w1m = cw1.transpose(2, 3, 1, 0).reshape(25, 16)
    o1 = _convmm_call(_im2col(x0p, 5, 5), w1m, cb1).reshape(B, 16, 16, 16)
    o1 = _avgpool2_nhwc(o1)
    w2m = cw2.transpose(2, 3, 1, 0).reshape(9 * 16, 32)
    o2 = _convmm_call(
        _im2col(jnp.pad(o1, ((0, 0), (1, 1), (1, 1), (0, 0))), 3, 3),
        w2m, cb2).reshape(B, 8, 8, 32)
    o2 = _avgpool2_nhwc(o2)
    w3m = cw3.transpose(2, 3, 1, 0).reshape(9 * 32, 32)
    o3 = _convmm_call(
        _im2col(jnp.pad(o2, ((0, 0), (1, 1), (1, 1), (0, 0))), 3, 3),
        w3m, cb3).reshape(B, 4, 4, 32)
    o3 = _avgpool2_nhwc(o3)
    o = o3.reshape(B, 128)

    jj = jnp.arange(128)
    G = ((jj[:, None] % 32) == (jj[None, :] % 32)).astype(jnp.float32) * 0.25
    gmp = jnp.tile(gamma, 4)[None, :]
    btp = jnp.tile(beta, 4)[None, :]
    a1 = lw1[0:F]
    a2 = lw1[F:2 * F]
    b1o = lw1[2 * F:].reshape(32, 2, 2, F).transpose(1, 2, 0, 3).reshape(F, F)

    return _final_call(ro, o, G, gmp, btp, a1, a2, b1o, lb1, lw2, lb2)


# speedup vs baseline: 24.9196x; 1.8313x over previous
import jax
import jax.numpy as jnp
from jax import lax
from jax.experimental import pallas as pl
from jax.experimental.pallas import tpu as pltpu
from jax.experimental.pallas import tpu_sc as plsc

N = 10000
E = 320000
B = 64
RATIO = 0.5
F = 128

NC = 2
NS = 16
NW = NC * NS
EPW = E // NW
CH = 80
SGE = 2000
CPS = SGE // CH
NSTAGE = EPW // SGE
NCHUNK = EPW // CH
NBUF = 5
NP = 10240
NSL = NP // NS


def _wid(c, s):
    return s * NC + c


def _deg_body(row_hbm, ew_hbm, zn_hbm, out_hbm, ewst, rowst, acc, dsem):
    c = lax.axis_index("c")
    s = lax.axis_index("s")
    pltpu.sync_copy(zn_hbm.at[pl.ds(s * NSL, NSL)], acc.at[pl.ds(s * NSL, NSL)])
    base = pl.multiple_of(_wid(c, s) * EPW, 8)
    pltpu.sync_copy(row_hbm.at[pl.ds(base, EPW)], rowst)
    plsc.subcore_barrier()

    def outer(t, carry):
        e0 = pl.multiple_of(base + t * SGE, 8)
        o0 = pl.multiple_of(t * SGE, 8)
        pltpu.sync_copy(ew_hbm.at[pl.ds(e0, SGE)], ewst)

        def chunk(k, carry2):
            o = pl.multiple_of(k * CH, 8)
            pltpu.async_copy(ewst.at[pl.ds(o, CH)],
                             acc.at[rowst.at[pl.ds(o0 + o, CH)]],
                             dsem, add=True)
            return carry2

        lax.fori_loop(0, CPS, chunk, carry)
        pltpu.make_async_copy(ewst, acc.at[pl.ds(0, SGE)], dsem).wait()
        return carry

    lax.fori_loop(0, NSTAGE, outer, 0)
    plsc.subcore_barrier()
    pltpu.sync_copy(acc.at[pl.ds(s * NSL, NSL)], out_hbm.at[c, pl.ds(s * NSL, NSL)])


_deg_call = pl.kernel(
    _deg_body,
    out_type=jax.ShapeDtypeStruct((NC, NP), jnp.float32),
    mesh=plsc.VectorSubcoreMesh(core_axis_name="c", subcore_axis_name="s"),
    compiler_params=pltpu.CompilerParams(needs_layout_passes=False),
    scratch_types=[
        pltpu.VMEM((SGE,), jnp.float32),
        pltpu.VMEM((EPW,), jnp.int32),
        pltpu.VMEM_SHARED((NP,), jnp.float32),
        pltpu.SemaphoreType.DMA,
    ],
)


def _agg_body(hd_hbm, col_hbm, row_hbm, ew_hbm, zf_hbm, out_hbm,
              colst, rowst, ewst, g0, g1, s0, s1,
              gm0, gm1, sm0, sm1, acc):
    g = [g0, g1]
    sv = [s0, s1]
    gsem = [gm0, gm1]
    ssem = [sm0, sm1]
    c = lax.axis_index("c")
    s = lax.axis_index("s")
    pltpu.sync_copy(zf_hbm.at[pl.ds(s * NSL, NSL)], acc.at[pl.ds(s * NSL, NSL)])
    base = pl.multiple_of(_wid(c, s) * EPW, 8)
    plsc.subcore_barrier()

    def group(t, carry):
        e0 = pl.multiple_of(base + t * SGE, 8)
        pltpu.sync_copy(col_hbm.at[pl.ds(e0, SGE)], colst)
        pltpu.sync_copy(row_hbm.at[pl.ds(e0, SGE)], rowst)
        pltpu.sync_copy(ew_hbm.at[pl.ds(e0, SGE)], ewst)
        for b in range(2):
            ob = pl.multiple_of(b * CH, 8)
            pltpu.async_copy(hd_hbm.at[colst.at[pl.ds(ob, CH)]], g[b], gsem[b])

        def slot(kg, carry2):
            for p in range(2):
                @pl.when(kg % 2 == p)
                def _():
                    pltpu.make_async_copy(
                        hd_hbm.at[pl.ds(0, CH)], g[p], gsem[p]).wait()

                    @pl.when(kg >= 2)
                    def _():
                        pltpu.make_async_copy(
                            sv[p], acc.at[pl.ds(0, CH)], ssem[p]).wait()

                    def scale(e, cc):
                        w16 = plsc.load_gather(
                            ewst, [jnp.zeros((16,), jnp.int32) + (kg * CH + e)])
                        for f in range(F // 16):
                            sv[p][e, pl.ds(f * 16, 16)] = (
                                g[p][e, pl.ds(f * 16, 16)] * w16)
                        return cc

                    lax.fori_loop(0, CH, scale, 0)
                    ok = pl.multiple_of(kg * CH, 8)
                    pltpu.async_copy(sv[p], acc.at[rowst.at[pl.ds(ok, CH)]],
                                     ssem[p], add=True)

                    @pl.when(kg + 2 < CPS)
                    def _():
                        on = pl.multiple_of((kg + 2) * CH, 8)
                        pltpu.async_copy(hd_hbm.at[colst.at[pl.ds(on, CH)]],
                                         g[p], gsem[p])
            return carry2

        lax.fori_loop(0, CPS, slot, carry)
        pltpu.make_async_copy(sv[0], acc.at[pl.ds(0, CH)], ssem[0]).wait()
        pltpu.make_async_copy(sv[1], acc.at[pl.ds(0, CH)], ssem[1]).wait()
        return carry

    lax.fori_loop(0, NSTAGE, group, 0)
    plsc.subcore_barrier()
    pltpu.sync_copy(acc.at[pl.ds(s * NSL, NSL)], out_hbm.at[c, pl.ds(s * NSL, NSL)])


_agg_call = pl.kernel(
    _agg_body,
    out_type=jax.ShapeDtypeStruct((NC, NP, F), jnp.float32),
    mesh=plsc.VectorSubcoreMesh(core_axis_name="c", subcore_axis_name="s"),
    compiler_params=pltpu.CompilerParams(needs_layout_passes=False),
    scratch_types=(
        [pltpu.VMEM((SGE,), jnp.int32),
         pltpu.VMEM((SGE,), jnp.int32),
         pltpu.VMEM((SGE,), jnp.float32)]
        + [pltpu.VMEM((CH, F), jnp.float32)] * 4
        + [pltpu.SemaphoreType.DMA] * 4
        + [pltpu.VMEM_SHARED((NP, F), jnp.float32)]
    ),
)


def _score_body(vg_hbm, col_hbm, row_hbm, ew_hbm, zn_hbm, out_hbm,
                colst, rowst, ewst, vgv, vout, acc, dsem):
    c = lax.axis_index("c")
    s = lax.axis_index("s")
    pltpu.sync_copy(zn_hbm.at[pl.ds(s * NSL, NSL)], acc.at[pl.ds(s * NSL, NSL)])
    pltpu.sync_copy(vg_hbm, vgv)
    base = pl.multiple_of(_wid(c, s) * EPW, 8)
    pltpu.sync_copy(col_hbm.at[pl.ds(base, EPW)], colst)
    pltpu.sync_copy(row_hbm.at[pl.ds(base, EPW)], rowst)
    plsc.subcore_barrier()

    def outer(t, carry):
        e0 = pl.multiple_of(base + t * SGE, 8)
        o0 = pl.multiple_of(t * SGE, 8)
        pltpu.sync_copy(ew_hbm.at[pl.ds(e0, SGE)], ewst)

        def comp(j, carry2):
            oj = pl.multiple_of(j * 16, 8)
            c16 = colst[pl.ds(o0 + oj, 16)]
            vv = plsc.load_gather(vgv, [c16])
            vout[pl.ds(oj, 16)] = vv * ewst[pl.ds(oj, 16)]
            return carry2

        lax.fori_loop(0, SGE // 16, comp, carry)

        def chunk(k, carry2):
            o = pl.multiple_of(k * CH, 8)
            pltpu.async_copy(vout.at[pl.ds(o, CH)],
                             acc.at[rowst.at[pl.ds(o0 + o, CH)]],
                             dsem, add=True)
            return carry2

        lax.fori_loop(0, CPS, chunk, carry)
        pltpu.make_async_copy(vout, acc.at[pl.ds(0, SGE)], dsem).wait()
        return carry

    lax.fori_loop(0, NSTAGE, outer, 0)
    plsc.subcore_barrier()
    pltpu.sync_copy(acc.at[pl.ds(s * NSL, NSL)], out_hbm.at[c, pl.ds(s * NSL, NSL)])


_score_call = pl.kernel(
    _score_body,
    out_type=jax.ShapeDtypeStruct((NC, NP), jnp.float32),
    mesh=plsc.VectorSubcoreMesh(core_axis_name="c", subcore_axis_name="s"),
    compiler_params=pltpu.CompilerParams(needs_layout_passes=False),
    scratch_types=[
        pltpu.VMEM((EPW,), jnp.int32),
        pltpu.VMEM((EPW,), jnp.int32),
        pltpu.VMEM((SGE,), jnp.float32),
        pltpu.VMEM((NP,), jnp.float32),
        pltpu.VMEM((SGE,), jnp.float32),
        pltpu.VMEM_SHARED((NP,), jnp.float32),
        pltpu.SemaphoreType.DMA,
    ],
)


def _ewn_body(mf_hbm, col_hbm, row_hbm, ew_hbm, zn_hbm, ewn_hbm, degp_hbm,
              colst, rowst, ewst, mv, vout, acc, dsem, wsem):
    c = lax.axis_index("c")
    s = lax.axis_index("s")
    pltpu.sync_copy(zn_hbm.at[pl.ds(s * NSL, NSL)], acc.at[pl.ds(s * NSL, NSL)])
    pltpu.sync_copy(mf_hbm, mv)
    base = pl.multiple_of(_wid(c, s) * EPW, 8)
    pltpu.sync_copy(col_hbm.at[pl.ds(base, EPW)], colst)
    pltpu.sync_copy(row_hbm.at[pl.ds(base, EPW)], rowst)
    plsc.subcore_barrier()

    def outer(t, carry):
        e0 = pl.multiple_of(base + t * SGE, 8)
        o0 = pl.multiple_of(t * SGE, 8)
        pltpu.sync_copy(ew_hbm.at[pl.ds(e0, SGE)], ewst)

        def comp(j, carry2):
            oj = pl.multiple_of(j * 16, 8)
            r16 = rowst[pl.ds(o0 + oj, 16)]
            c16 = colst[pl.ds(o0 + oj, 16)]
            mr = plsc.load_gather(mv, [r16])
            mc = plsc.load_gather(mv, [c16])
            vout[pl.ds(oj, 16)] = ewst[pl.ds(oj, 16)] * mr * mc
            return carry2

        lax.fori_loop(0, SGE // 16, comp, carry)
        pltpu.async_copy(vout, ewn_hbm.at[pl.ds(e0, SGE)], wsem)

        def chunk(k, carry2):
            o = pl.multiple_of(k * CH, 8)
            pltpu.async_copy(vout.at[pl.ds(o, CH)],
                             acc.at[rowst.at[pl.ds(o0 + o, CH)]],
                             dsem, add=True)
            return carry2

        lax.fori_loop(0, CPS, chunk, carry)
        pltpu.make_async_copy(vout, acc.at[pl.ds(0, SGE)], dsem).wait()
        pltpu.make_async_copy(vout, ewn_hbm.at[pl.ds(0, SGE)], wsem).wait()
        return carry

    lax.fori_loop(0, NSTAGE, outer, 0)
    plsc.subcore_barrier()
    pltpu.sync_copy(acc.at[pl.ds(s * NSL, NSL)], degp_hbm.at[c, pl.ds(s * NSL, NSL)])


_ewn_call = pl.kernel(
    _ewn_body,
    out_type=[
        jax.ShapeDtypeStruct((E,), jnp.float32),
        jax.ShapeDtypeStruct((NC, NP), jnp.float32),
    ],
    mesh=plsc.VectorSubcoreMesh(core_axis_name="c", subcore_axis_name="s"),
    compiler_params=pltpu.CompilerParams(needs_layout_passes=False),
    scratch_types=[
        pltpu.VMEM((EPW,), jnp.int32),
        pltpu.VMEM((EPW,), jnp.int32),
        pltpu.VMEM((SGE,), jnp.float32),
        pltpu.VMEM((NP,), jnp.float32),
        pltpu.VMEM((SGE,), jnp.float32),
        pltpu.VMEM_SHARED((NP,), jnp.float32),
        pltpu.SemaphoreType.DMA,
        pltpu.SemaphoreType.DMA,
    ],
)


TI = 1024
NBI = NP // TI
TJ = 2048
NBJ = NP // TJ


def _dis_body(degp_ref, dis_ref):
    d = degp_ref[...]
    dis_ref[...] = lax.rsqrt(jnp.maximum(d[0:1, :] + d[1:2, :] + 1.0, 1e-12))


def _dis_call(degP):
    return pl.pallas_call(
        _dis_body,
        out_shape=jax.ShapeDtypeStruct((1, NP), jnp.float32),
    )(degP)


def _hscale_body(dis_ref, h_ref, hd_ref):
    hd_ref[...] = dis_ref[...] * h_ref[...]


def _hscale_call(dis_col, h):
    return pl.pallas_call(
        _hscale_body,
        grid=(NBI,),
        in_specs=[
            pl.BlockSpec((TI, 1), lambda i: (i, 0)),
            pl.BlockSpec((TI, F), lambda i: (i, 0)),
        ],
        out_specs=pl.BlockSpec((TI, F), lambda i: (i, 0)),
        out_shape=jax.ShapeDtypeStruct((NP, F), jnp.float32),
    )(dis_col, h)


def _gcnmm_body(p_ref, hd_ref, dis_ref, w_ref, b_ref, ws_ref, hp_ref, vg_ref):
    dis = dis_ref[...]
    p = p_ref[...]
    g = dis * (p[0] + p[1] + hd_ref[...])
    hp = jax.nn.relu(
        jnp.dot(g, w_ref[...], preferred_element_type=jnp.float32) + b_ref[...])
    hp_ref[...] = hp
    vg_ref[...] = dis * jnp.dot(hp, ws_ref[...],
                                preferred_element_type=jnp.float32)


def _gcnmm_call(P, hd, dis_col, W, b, Ws):
    return pl.pallas_call(
        _gcnmm_body,
        grid=(NBI,),
        in_specs=[
            pl.BlockSpec((NC, TI, F), lambda i: (0, i, 0)),
            pl.BlockSpec((TI, F), lambda i: (i, 0)),
            pl.BlockSpec((TI, 1), lambda i: (i, 0)),
            pl.BlockSpec((F, F), lambda i: (0, 0)),
            pl.BlockSpec((1, F), lambda i: (0, 0)),
            pl.BlockSpec((F, 1), lambda i: (0, 0)),
        ],
        out_specs=[
            pl.BlockSpec((TI, F), lambda i: (i, 0)),
            pl.BlockSpec((TI, 1), lambda i: (i, 0)),
        ],
        out_shape=[
            jax.ShapeDtypeStruct((NP, F), jnp.float32),
            jax.ShapeDtypeStruct((NP, 1), jnp.float32),
        ],
    )(P, hd, dis_col, W, b, Ws)


def _smask_body(sp_ref, vg_ref, dis_ref, mf_ref, bs_ref, sm_ref):
    sp = sp_ref[...]
    score = dis_ref[...] * (sp[0:1, :] + sp[1:2, :] + vg_ref[...]) + bs_ref[0, 0]
    sm_ref[...] = jnp.where(mf_ref[...] > 0.5, score, -1e30)


def _smask_call(sP, vg_row, dis_row, mf_row, bs):
    return pl.pallas_call(
        _smask_body,
        out_shape=jax.ShapeDtypeStruct((1, NP), jnp.float32),
    )(sP, vg_row, dis_row, mf_row, bs)


def _rank_body(si_ref, sj_ref, bi_ref, bj_ref, mfj_ref, rank_ref, alive_ref):
    i = pl.program_id(0)
    j = pl.program_id(1)
    bi = bi_ref[...]
    bj = bj_ref[...]

    @pl.when(j == 0)
    def _():
        rank_ref[...] = jnp.zeros_like(rank_ref)

    @pl.when(jnp.logical_and(i == 0, j == 0))
    def _():
        alive_ref[...] = jnp.zeros_like(alive_ref)

    overlap = jnp.logical_and(bj[0, TJ - 1] >= bi[0, 0],
                              bj[0, 0] <= bi[TI - 1, 0])

    @pl.when(overlap)
    def _():
        si = si_ref[...]
        sj = sj_ref[...]
        ig = i * TI + lax.broadcasted_iota(jnp.int32, (TI, 1), 0)
        jg = j * TJ + lax.broadcasted_iota(jnp.int32, (1, TJ), 1)
        cond = (sj > si) | ((sj == si) & (jg < ig))
        add = ((bj == bi) & cond).astype(jnp.float32)
        rank_ref[...] += jnp.sum(add, axis=1, keepdims=True)

    @pl.when(i == 0)
    def _():
        oh = (bj == lax.broadcasted_iota(jnp.int32, (B, 1), 0)).astype(jnp.float32)
        alive_ref[...] += jnp.sum(oh * mfj_ref[...], axis=1, keepdims=True)


def _rank_call(sm_col, sm_row, b_col, b_row, mf_row):
    return pl.pallas_call(
        _rank_body,
        grid=(NBI, NBJ),
        in_specs=[
            pl.BlockSpec((TI, 1), lambda i, j: (i, 0)),
            pl.BlockSpec((1, TJ), lambda i, j: (0, j)),
            pl.BlockSpec((TI, 1), lambda i, j: (i, 0)),
            pl.BlockSpec((1, TJ), lambda i, j: (0, j)),
            pl.BlockSpec((1, TJ), lambda i, j: (0, j)),
        ],
        out_specs=[
            pl.BlockSpec((TI, 1), lambda i, j: (i, 0)),
            pl.BlockSpec((B, 1), lambda i, j: (0, 0)),
        ],
        out_shape=[
            jax.ShapeDtypeStruct((NP, 1), jnp.float32),
            jax.ShapeDtypeStruct((B, 1), jnp.float32),
        ],
    )(sm_col, sm_row, b_col, b_row, mf_row)


def _pool_body(rank_ref, alive_ref, sm_ref, mf_ref, b_ref, hp_ref,
               xn_ref, nmf_ref, sacc_ref, cnt_ref, mx_ref):
    i = pl.program_id(0)
    kb = jnp.ceil(RATIO * alive_ref[...])
    bi = b_ref[...]
    oh = (bi == lax.broadcasted_iota(jnp.int32, (1, B), 1)).astype(jnp.float32)
    kk = jnp.dot(oh, kb, preferred_element_type=jnp.float32)
    nm = jnp.logical_and(rank_ref[...] < kk, mf_ref[...] > 0.5)
    nmf = nm.astype(jnp.float32)
    xn = hp_ref[...] * jnp.tanh(sm_ref[...]) * nmf
    xn_ref[...] = xn
    nmf_ref[...] = nmf

    @pl.when(i == 0)
    def _():
        sacc_ref[...] = jnp.zeros_like(sacc_ref)
        cnt_ref[...] = jnp.zeros_like(cnt_ref)
        mx_ref[...] = jnp.full_like(mx_ref, -1e30)

    contract = (((0,), (0,)), ((), ()))
    sacc_ref[...] += lax.dot_general(oh, xn, contract,
                                     preferred_element_type=jnp.float32)
    cnt_ref[...] += lax.dot_general(oh, nmf, contract,
                                    preferred_element_type=jnp.float32)
    blo = bi[0, 0]
    bhi = bi[TI - 1, 0]
    rows = []
    for bb in range(B):
        xm = jnp.where(jnp.logical_and(nm, bi == bb), xn, -1e30)
        rows.append(jnp.max(xm, axis=0, keepdims=True))
    upd = jnp.concatenate(rows, axis=0)
    bidx = lax.broadcasted_iota(jnp.int32, (B, 1), 0)
    inblk = jnp.logical_and(bidx >= blo, bidx <= bhi)
    mx_ref[...] = jnp.maximum(mx_ref[...], jnp.where(inblk, upd, -1e30))


def _pool_call(rank, alive, sm_col, mf_col, b_col, hp):
    return pl.pallas_call(
        _pool_body,
        grid=(NBI,),
        in_specs=[
            pl.BlockSpec((TI, 1), lambda i: (i, 0)),
            pl.BlockSpec((B, 1), lambda i: (0, 0)),
            pl.BlockSpec((TI, 1), lambda i: (i, 0)),
            pl.BlockSpec((TI, 1), lambda i: (i, 0)),
            pl.BlockSpec((TI, 1), lambda i: (i, 0)),
            pl.BlockSpec((TI, F), lambda i: (i, 0)),
        ],
        out_specs=[
            pl.BlockSpec((TI, F), lambda i: (i, 0)),
            pl.BlockSpec((TI, 1), lambda i: (i, 0)),
            pl.BlockSpec((B, F), lambda i: (0, 0)),
            pl.BlockSpec((B, 1), lambda i: (0, 0)),
            pl.BlockSpec((B, F), lambda i: (0, 0)),
        ],
        out_shape=[
            jax.ShapeDtypeStruct((NP, F), jnp.float32),
            jax.ShapeDtypeStruct((NP, 1), jnp.float32),
            jax.ShapeDtypeStruct((B, F), jnp.float32),
            jax.ShapeDtypeStruct((B, 1), jnp.float32),
            jax.ShapeDtypeStruct((B, F), jnp.float32),
        ],
    )(rank, alive, sm_col, mf_col, b_col, hp)


def _convmm_body(x_ref, w_ref, b_ref, o_ref):
    o_ref[...] = jax.nn.relu(
        jnp.dot(x_ref[...], w_ref[...], preferred_element_type=jnp.float32)
        + b_ref[...])


def _convmm_call(patches, wmat, bvec):
    m, k = patches.shape
    co = wmat.shape[1]
    return pl.pallas_call(
        _convmm_body,
        out_shape=jax.ShapeDtypeStruct((m, co), jnp.float32),
    )(patches, wmat, bvec[None, :])


def _final_body(mx1, s1, c1, mx2, s2, c2, mx3, s3, c3, o_ref, g_ref,
                gmp_ref, btp_ref, a1_ref, a2_ref, b1o_ref, lb1_ref,
                lw2_ref, lb2_ref, out_ref):
    o = o_ref[...]
    mu = jnp.mean(o, axis=0, keepdims=True)
    m2 = jnp.mean(o * o, axis=0, keepdims=True)
    muc = jnp.dot(mu, g_ref[...], preferred_element_type=jnp.float32)
    m2c = jnp.dot(m2, g_ref[...], preferred_element_type=jnp.float32)
    var = m2c - muc * muc
    on = (o - muc) / jnp.sqrt(var + 1e-5) * gmp_ref[...] + btp_ref[...]

    z = lb1_ref[...] + jnp.dot(on, b1o_ref[...],
                               preferred_element_type=jnp.float32)
    for mx, s, c in ((mx1, s1, c1), (mx2, s2, c2), (mx3, s3, c3)):
        cnt = c[...]
        mxf = jnp.where(cnt > 0, mx[...], 0.0)
        mean = s[...] / jnp.maximum(cnt, 1.0)
        z += jnp.dot(mxf, a1_ref[...], preferred_element_type=jnp.float32)
        z += jnp.dot(mean, a2_ref[...], preferred_element_type=jnp.float32)
    z = jax.nn.relu(z)
    out_ref[...] = jax.nn.sigmoid(
        jnp.dot(z, lw2_ref[...], preferred_element_type=jnp.float32)
        + lb2_ref[...])


def _final_call(ro, o, G, gmp, btp, a1, a2, b1o, lb1, lw2, lb2):
    (mx1, s1, c1), (mx2, s2, c2), (mx3, s3, c3) = ro
    return pl.pallas_call(
        _final_body,
        out_shape=jax.ShapeDtypeStruct((B, 1), jnp.float32),
    )(mx1, s1, c1, mx2, s2, c2, mx3, s3, c3, o, G, gmp, btp,
      a1, a2, b1o, lb1[None, :], lw2, lb2[None, :])


def _im2col(xp, kh, kw):
    b, hp_, wp_, c = xp.shape
    h = hp_ - kh + 1
    w = wp_ - kw + 1
    cols = []
    for dy in range(kh):
        for dx in range(kw):
            cols.append(xp[:, dy:dy + h, dx:dx + w, :])
    pt = jnp.stack(cols, axis=3)
    return pt.reshape(b * h * w, kh * kw * c)


def _avgpool2_nhwc(x):
    b, h, w, c = x.shape
    return x.reshape(b, h // 2, 2, w // 2, 2, c).mean(axis=(2, 4))


def kernel(x, pos, edge_attr, strata_data, W1, b1, Ws1, bs1, W2, b2, Ws2, bs2,
           W3, b3, Ws3, bs3, cw1, cb1, cw2, cb2, cw3, cb3, gamma, beta,
           lw1, lb1, lw2, lb2, edge_index, batch):
    row = edge_index[0]
    col = edge_index[1]
    ew = edge_attr
    zn = jnp.zeros((NP,), jnp.float32)
    zf = jnp.zeros((NP, F), jnp.float32)

    h = jnp.zeros((NP, F), jnp.float32).at[:N].set(
        jnp.concatenate([x, pos], axis=1))
    batch_p = jnp.full((NP,), B, jnp.int32).at[:N].set(batch)
    b_col = batch_p.reshape(NP, 1)
    b_row = batch_p.reshape(1, NP)
    mf_col = jnp.zeros((NP, 1), jnp.float32).at[:N].set(1.0)

    degP = _deg_call(row, ew, zn)
    dis_row = _dis_call(degP)
    dis_col = dis_row.reshape(NP, 1)

    ro = []
    weights = [(W1, b1, Ws1, bs1), (W2, b2, Ws2, bs2), (W3, b3, Ws3, bs3)]
    for r in range(3):
        W, b, Ws, bs = weights[r]
        hd = _hscale_call(dis_col, h)
        P = _agg_call(hd, col, row, ew, zf)
        hp, vg_col = _gcnmm_call(P, hd, dis_col, W, b[None, :], Ws)
        sP = _score_call(vg_col.reshape(NP), col, row, ew, zn)
        sm_row = _smask_call(sP, vg_col.reshape(1, NP), dis_row,
                             mf_col.reshape(1, NP), bs[None, :])
        sm_col = sm_row.reshape(NP, 1)
        rank, alive = _rank_call(sm_col, sm_row, b_col, b_row,
                                 mf_col.reshape(1, NP))
        xn, nmf_col, sacc, cnt, mx = _pool_call(rank, alive, sm_col,
                                                mf_col, b_col, hp)
        ro.append((mx, sacc, cnt))
        h = xn
        mf_col = nmf_col
        if r < 2:
            ew, degP = _ewn_call(nmf_col.reshape(NP), col, row, ew, zn)
            dis_row = _dis_call(degP)
            dis_col = dis_row.reshape(NP, 1)

    x0p = jnp.pad(strata_data.transpose(0, 2, 3, 1),
                  ((0, 0), (2, 2), (2, 2), (0, 0)))
    w1m = cw1.transpose(2, 3, 1, 0).reshape(25, 16)
    o1 = _convmm_call(_im2col(x0p, 5, 5), w1m, cb1).reshape(B, 16, 16, 16)
    o1 = _avgpool2_nhwc(o1)
    w2m = cw2.transpose(2, 3, 1, 0).reshape(9 * 16, 32)
    o2 = _convmm_call(
        _im2col(jnp.pad(o1, ((0, 0), (1, 1), (1, 1), (0, 0))), 3, 3),
        w2m, cb2).reshape(B, 8, 8, 32)
    o2 = _avgpool2_nhwc(o2)
    w3m = cw3.transpose(2, 3, 1, 0).reshape(9 * 32, 32)
    o3 = _convmm_call(
        _im2col(jnp.pad(o2, ((0, 0), (1, 1), (1, 1), (0, 0))), 3, 3),
        w3m, cb3).reshape(B, 4, 4, 32)
    o3 = _avgpool2_nhwc(o3)
    o = o3.reshape(B, 128)

    jj = jnp.arange(128)
    G = ((jj[:, None] % 32) == (jj[None, :] % 32)).astype(jnp.float32) * 0.25
    gmp = jnp.tile(gamma, 4)[None, :]
    btp = jnp.tile(beta, 4)[None, :]
    a1 = lw1[0:F]
    a2 = lw1[F:2 * F]
    b1o = lw1[2 * F:].reshape(32, 2, 2, F).transpose(1, 2, 0, 3).reshape(F, F)

    return _final_call(ro, o, G, gmp, btp, a1, a2, b1o, lb1, lw2, lb2)
